# baseline (speedup 1.0000x reference)
import functools
import math

import jax
import jax.numpy as jnp
import numpy as np
from jax import lax
from jax.experimental import pallas as pl
from jax.experimental.pallas import tpu as pltpu
from jax.experimental.pallas import tpu_sc as plsc

D_MODEL = 128
SCALE = np.float32(math.sqrt(128.0))

NC = 2
NS = 16
NW = NC * NS
LANES = 16
W = 400
NBUF = 2


@functools.lru_cache(maxsize=None)
def _build(n_total: int):
    assert n_total % (NW * W) == 0
    b_per_w = n_total // NW
    n_chunks = b_per_w // W
    assert n_chunks % NBUF == 0 and n_chunks >= 2 * NBUF
    mesh = plsc.VectorSubcoreMesh(core_axis_name="c", subcore_axis_name="s")

    bufs_t = [pltpu.VMEM((W, D_MODEL), jnp.float32) for _ in range(NBUF)]
    gsems_t = [pltpu.SemaphoreType.DMA for _ in range(NBUF)]
    ssems_t = [pltpu.SemaphoreType.DMA for _ in range(NBUF)]

    @functools.partial(
        pl.kernel,
        mesh=mesh,
        out_type=jax.ShapeDtypeStruct((n_total, D_MODEL), jnp.float32),
        scratch_types=[pltpu.VMEM((b_per_w,), jnp.int32)]
        + bufs_t + gsems_t + ssems_t,
    )
    def emb(table_hbm, idx_hbm, out_hbm, idx_v, *rest):
        bufs = rest[:NBUF]
        gsems = rest[NBUF:2 * NBUF]
        ssems = rest[2 * NBUF:]
        wid = lax.axis_index("s") * NC + lax.axis_index("c")
        base = wid * b_per_w
        pltpu.sync_copy(idx_hbm.at[pl.ds(base, b_per_w)], idx_v)

        def start_gather(c, b):
            pltpu.async_copy(table_hbm.at[idx_v.at[pl.ds(c * W, W)]],
                             bufs[b], gsems[b])

        def wait_dma(b, sem):
            pltpu.make_async_copy(table_hbm.at[pl.ds(0, W)], bufs[b], sem).wait()

        def scale(buf):
            @pl.loop(0, W, step=2)
            def _(r):
                for rr in range(2):
                    for c in range(0, D_MODEL, LANES):
                        slc = (r + rr, pl.ds(c, LANES))
                        buf.at[slc][...] = buf.at[slc][...] * SCALE

        for b in range(NBUF):
            start_gather(b, b)

        @pl.loop(0, n_chunks, step=NBUF)
        def _(g):
            for b in range(NBUF):
                c = g + b
                wait_dma(b, gsems[b])
                scale(bufs[b])
                pltpu.async_copy(bufs[b], out_hbm.at[pl.ds(base + c * W, W)],
                                 ssems[b])

                pb = (b - 1) % NBUF
                pc = c - 1

                @pl.when((pc >= 0) & (pc + NBUF < n_chunks))
                def _():
                    wait_dma(pb, ssems[pb])
                    start_gather(pc + NBUF, pb)

        for b in range(NBUF):
            wait_dma(b, ssems[b])

    return emb


def kernel(x, table):
    b, s = x.shape
    n = b * s
    idx = x.reshape(n).astype(jnp.int32)
    out = _build(n)(table, idx)
    return out.reshape(b, s, D_MODEL)


# speedup vs baseline: 1.1505x; 1.1505x over previous
import functools
import math

import jax
import jax.numpy as jnp
import numpy as np
from jax import lax
from jax.experimental import pallas as pl
from jax.experimental.pallas import tpu as pltpu
from jax.experimental.pallas import tpu_sc as plsc

D_MODEL = 128
SCALE = np.float32(math.sqrt(128.0))

NC = 2
NS = 16
NW = NC * NS
LANES = 16
W = 160
NBUF = 5


@functools.lru_cache(maxsize=None)
def _build(n_total: int):
    assert n_total % (NW * W) == 0
    b_per_w = n_total // NW
    n_chunks = b_per_w // W
    assert n_chunks % NBUF == 0 and n_chunks >= 2 * NBUF
    mesh = plsc.VectorSubcoreMesh(core_axis_name="c", subcore_axis_name="s")

    bufs_t = [pltpu.VMEM((W, D_MODEL), jnp.float32) for _ in range(NBUF)]
    gsems_t = [pltpu.SemaphoreType.DMA for _ in range(NBUF)]
    ssems_t = [pltpu.SemaphoreType.DMA for _ in range(NBUF)]

    @functools.partial(
        pl.kernel,
        mesh=mesh,
        out_type=jax.ShapeDtypeStruct((n_total, D_MODEL), jnp.float32),
        scratch_types=[pltpu.VMEM((b_per_w,), jnp.int32)]
        + bufs_t + gsems_t + ssems_t,
    )
    def emb(table_hbm, idx_hbm, out_hbm, idx_v, *rest):
        bufs = rest[:NBUF]
        gsems = rest[NBUF:2 * NBUF]
        ssems = rest[2 * NBUF:]
        wid = lax.axis_index("s") * NC + lax.axis_index("c")
        base = wid * b_per_w
        pltpu.sync_copy(idx_hbm.at[pl.ds(base, b_per_w)], idx_v)

        def start_gather(c, b):
            pltpu.async_copy(table_hbm.at[idx_v.at[pl.ds(c * W, W)]],
                             bufs[b], gsems[b])

        def wait_dma(b, sem):
            pltpu.make_async_copy(table_hbm.at[pl.ds(0, W)], bufs[b], sem).wait()

        def scale(buf):
            @pl.loop(0, W, step=2)
            def _(r):
                for rr in range(2):
                    for c in range(0, D_MODEL, LANES):
                        slc = (r + rr, pl.ds(c, LANES))
                        buf.at[slc][...] = buf.at[slc][...] * SCALE

        for b in range(NBUF):
            start_gather(b, b)

        @pl.loop(0, n_chunks, step=NBUF)
        def _(g):
            for b in range(NBUF):
                c = g + b
                wait_dma(b, gsems[b])
                scale(bufs[b])
                pltpu.async_copy(bufs[b], out_hbm.at[pl.ds(base + c * W, W)],
                                 ssems[b])

                pb = (b - 1) % NBUF
                pc = c - 1

                @pl.when((pc >= 0) & (pc + NBUF < n_chunks))
                def _():
                    wait_dma(pb, ssems[pb])
                    start_gather(pc + NBUF, pb)

        for b in range(NBUF):
            wait_dma(b, ssems[b])

    return emb


def kernel(x, table):
    b, s = x.shape
    n = b * s
    idx = x.reshape(n).astype(jnp.int32)
    out = _build(n)(table, idx)
    return out.reshape(b, s, D_MODEL)
